# baseline (speedup 1.0000x reference)
import functools

import jax
import jax.numpy as jnp
from jax import lax
from jax.experimental import pallas as pl
from jax.experimental.pallas import tpu as pltpu
from jax.experimental.pallas import tpu_sc as plsc

V = 10000
D = 128
P = D // 2
B = 320000
NC, NS = 2, 16
NW = NC * NS
E_PER_W = B // NW
C = 80
NCH = E_PER_W // C
GRP = C // 16
NBUF = 4


def _body(table, src_i, dst_i, out, idx_s, idx_d, rows_s, rows_d, out_v,
          table_sh, *sems_flat):
    sid = lax.axis_index("s")
    wid = lax.axis_index("c") * NS + sid

    pltpu.sync_copy(table.at[pl.ds(sid * (V // NS), V // NS)],
                    table_sh.at[pl.ds(sid * (V // NS), V // NS)])
    pltpu.sync_copy(src_i.at[wid], idx_s)
    pltpu.sync_copy(dst_i.at[wid], idx_d)
    plsc.subcore_barrier()

    sems = tuple(zip(sems_flat[:NBUF], sems_flat[NBUF:]))

    def start(g, b):
        ss, sd = sems[b]
        pltpu.async_copy(table.at[idx_s.at[g]], rows_s.at[b], ss)
        pltpu.async_copy(table.at[idx_d.at[g]], rows_d.at[b], sd)

    def compute(g, b):
        ss, sd = sems[b]
        pltpu.make_async_copy(table.at[idx_s.at[g]], rows_s.at[b], ss).wait()
        pltpu.make_async_copy(table.at[idx_d.at[g]], rows_d.at[b], sd).wait()
        rs = rows_s.at[b]
        rd = rows_d.at[b]
        lanes = jnp.arange(16, dtype=jnp.int32)
        himask = jnp.int32(-65536)
        for grp in range(GRP):
            eidx = lanes + (grp * 16)

            def dbody(i, accs):
                nxt = []
                for j, acc in enumerate(accs):
                    pp = (lanes + (i * 8 + j)) & (P - 1)
                    si = plsc.load_gather(rs, [eidx, pp])
                    di = plsc.load_gather(rd, [eidx, pp])
                    s_lo = plsc.bitcast(si << 16, jnp.float32)
                    d_lo = plsc.bitcast(di << 16, jnp.float32)
                    s_hi = plsc.bitcast(si & himask, jnp.float32)
                    d_hi = plsc.bitcast(di & himask, jnp.float32)
                    nxt.append(acc + (s_lo * d_lo + s_hi * d_hi))
                return tuple(nxt)

            zero = jnp.zeros((16,), jnp.float32)
            accs = lax.fori_loop(0, P // 8, dbody, (zero,) * 8)
            a = ((accs[0] + accs[1]) + (accs[2] + accs[3])) + (
                (accs[4] + accs[5]) + (accs[6] + accs[7]))
            res = 1.0 / (1.0 + jnp.exp(-a))
            out_v[pl.ds(g * C + grp * 16, 16)] = res

    for b in range(NBUF):
        start(b, b)

    def ring(q, carry):
        for b in range(NBUF):
            g = NBUF * q + b
            compute(g, b)

            @pl.when(g + NBUF < NCH)
            def _():
                start(g + NBUF, b)

        return carry

    lax.fori_loop(0, NCH // NBUF, ring, 0)
    for g in range((NCH // NBUF) * NBUF, NCH):
        compute(g, g % NBUF)

    pltpu.sync_copy(out_v, out.at[pl.ds(wid * E_PER_W, E_PER_W)])


@functools.partial(jax.jit, donate_argnums=())
def _decode(table_pk, src_i, dst_i):
    run = functools.partial(
        pl.kernel,
        out_type=jax.ShapeDtypeStruct((B,), jnp.float32),
        mesh=plsc.VectorSubcoreMesh(core_axis_name="c", subcore_axis_name="s"),
        compiler_params=pltpu.CompilerParams(
            needs_layout_passes=False, use_tc_tiling_on_sc=False),
        scratch_types=[
            pltpu.VMEM((NCH, C), jnp.int32),
            pltpu.VMEM((NCH, C), jnp.int32),
            pltpu.VMEM((NBUF, C, P), jnp.int32),
            pltpu.VMEM((NBUF, C, P), jnp.int32),
            pltpu.VMEM((E_PER_W,), jnp.float32),
            pltpu.VMEM_SHARED((V, P), jnp.int32),
        ] + [pltpu.SemaphoreType.DMA] * (2 * NBUF),
    )(_body)
    return run(table_pk, src_i, dst_i)


def kernel(quantized_latent_embedding, edge_index):
    table_pk = lax.bitcast_convert_type(
        quantized_latent_embedding.astype(jnp.bfloat16).reshape(V, P, 2),
        jnp.int32)
    src_i = edge_index[0].astype(jnp.int32).reshape(NW, NCH, C)
    dst_i = edge_index[1].astype(jnp.int32).reshape(NW, NCH, C)
    return _decode(table_pk, src_i, dst_i)


# speedup vs baseline: 1.0901x; 1.0901x over previous
import functools

import jax
import jax.numpy as jnp
from jax import lax
from jax.experimental import pallas as pl
from jax.experimental.pallas import tpu as pltpu
from jax.experimental.pallas import tpu_sc as plsc

V = 10000
D = 128
P = D // 2
B = 320000
NC, NS = 2, 16
NW = NC * NS
E_PER_W = B // NW
C = 400
NCH = E_PER_W // C
GRP = C // 16
NBUF = 2


def _body(table, src_i, dst_i, out, idx_s, idx_d, rows_s, rows_d, out_b,
          *sems_flat):
    sid = lax.axis_index("s")
    wid = lax.axis_index("c") * NS + sid
    out_base = wid * E_PER_W

    pltpu.sync_copy(src_i.at[wid], idx_s)
    pltpu.sync_copy(dst_i.at[wid], idx_d)

    sem_s = sems_flat[:NBUF]
    sem_d = sems_flat[NBUF:NBUF * 2]
    sem_o = sems_flat[NBUF * 2:]

    def start(g, b):
        pltpu.async_copy(table.at[idx_s.at[g]], rows_s.at[b], sem_s[b])
        pltpu.async_copy(table.at[idx_d.at[g]], rows_d.at[b], sem_d[b])

    def compute(g, b):
        pltpu.make_async_copy(table.at[idx_s.at[g]], rows_s.at[b],
                              sem_s[b]).wait()
        pltpu.make_async_copy(table.at[idx_d.at[g]], rows_d.at[b],
                              sem_d[b]).wait()

        @pl.when(g >= NBUF)
        def _():
            pltpu.make_async_copy(
                out_b.at[b],
                out.at[pl.ds(out_base + (g - NBUF) * C, C)],
                sem_o[b]).wait()

        rs = rows_s.at[b]
        rd = rows_d.at[b]
        ob = out_b.at[b]
        lanes = jnp.arange(16, dtype=jnp.int32)
        himask = jnp.int32(-65536)

        def gbody(grp, carry):
            eidx = lanes + grp * 16

            def dbody(i, accs):
                nxt = []
                for j, acc in enumerate(accs):
                    pp = (lanes + (i * 8 + j)) & (P - 1)
                    si = plsc.load_gather(rs, [eidx, pp])
                    di = plsc.load_gather(rd, [eidx, pp])
                    s_lo = plsc.bitcast(si << 16, jnp.float32)
                    d_lo = plsc.bitcast(di << 16, jnp.float32)
                    s_hi = plsc.bitcast(si & himask, jnp.float32)
                    d_hi = plsc.bitcast(di & himask, jnp.float32)
                    nxt.append(acc + (s_lo * d_lo + s_hi * d_hi))
                return tuple(nxt)

            zero = jnp.zeros((16,), jnp.float32)
            accs = lax.fori_loop(0, P // 8, dbody, (zero,) * 8)
            a = ((accs[0] + accs[1]) + (accs[2] + accs[3])) + (
                (accs[4] + accs[5]) + (accs[6] + accs[7]))
            res = 1.0 / (1.0 + jnp.exp(-a))
            ob[pl.ds(grp * 16, 16)] = res
            return carry

        lax.fori_loop(0, GRP, gbody, 0)
        pltpu.async_copy(ob, out.at[pl.ds(out_base + g * C, C)], sem_o[b])

    for b in range(NBUF):
        start(b, b)

    def ring(q, carry):
        for b in range(NBUF):
            g = NBUF * q + b
            compute(g, b)

            @pl.when(g + NBUF < NCH)
            def _():
                start(g + NBUF, b)

        return carry

    lax.fori_loop(0, NCH // NBUF, ring, 0)
    for g in range((NCH // NBUF) * NBUF, NCH):
        compute(g, g % NBUF)

    for g in range(max(NCH - NBUF, 0), NCH):
        b = g % NBUF
        pltpu.make_async_copy(out_b.at[b],
                              out.at[pl.ds(out_base + g * C, C)],
                              sem_o[b]).wait()


@functools.partial(jax.jit, donate_argnums=())
def _decode(table_pk, src_i, dst_i):
    run = functools.partial(
        pl.kernel,
        out_type=jax.ShapeDtypeStruct((B,), jnp.float32),
        mesh=plsc.VectorSubcoreMesh(core_axis_name="c", subcore_axis_name="s"),
        compiler_params=pltpu.CompilerParams(
            needs_layout_passes=False, use_tc_tiling_on_sc=False),
        scratch_types=[
            pltpu.VMEM((NCH, C), jnp.int32),
            pltpu.VMEM((NCH, C), jnp.int32),
            pltpu.VMEM((NBUF, C, P), jnp.int32),
            pltpu.VMEM((NBUF, C, P), jnp.int32),
            pltpu.VMEM((NBUF, C), jnp.float32),
        ] + [pltpu.SemaphoreType.DMA] * (3 * NBUF),
    )(_body)
    return run(table_pk, src_i, dst_i)


def kernel(quantized_latent_embedding, edge_index):
    table_pk = lax.bitcast_convert_type(
        quantized_latent_embedding.astype(jnp.bfloat16).reshape(V, P, 2),
        jnp.int32)
    src_i = edge_index[0].astype(jnp.int32).reshape(NW, NCH, C)
    dst_i = edge_index[1].astype(jnp.int32).reshape(NW, NCH, C)
    return _decode(table_pk, src_i, dst_i)
